# baseline (speedup 1.0000x reference)
import functools

import jax
import jax.numpy as jnp
from jax import lax
from jax.experimental import pallas as pl
from jax.experimental.pallas import tpu as pltpu
from jax.experimental.pallas import tpu_sc as plsc

N = 10000
D = 128
E = 320000

NC = 2
NS = 16
NT = NC * NS
EPT = E // NT
C = 100
NCH = EPT // C
RPT_A = 632
RPT_B = N - (NS - 1) * RPT_A


NBUF = 3
IB = 4


def _agg_body(h_hbm, zeros_hbm, src_hbm, dst_hbm, out_hbm,
              acc, src_v, dst_v, rows, isems, gsems):
    cid = lax.axis_index("core")
    sid = lax.axis_index("subcore")
    t = cid * NS + sid

    row0 = sid * RPT_A
    init_src = [h_hbm, zeros_hbm]
    for c in range(NC):
        @pl.when((cid == c) & (sid < NS - 1))
        def _(src=init_src[c]):
            pltpu.sync_copy(src.at[pl.ds(row0, RPT_A)],
                            acc.at[pl.ds(row0, RPT_A)])

        @pl.when((cid == c) & (sid == NS - 1))
        def _(src=init_src[c]):
            pltpu.sync_copy(src.at[pl.ds(row0, RPT_B)],
                            acc.at[pl.ds(row0, RPT_B)])

    plsc.subcore_barrier()

    def _load_idx(ch, slot):
        pltpu.async_copy(src_hbm.at[t, ch], src_v.at[slot], isems.at[slot])
        pltpu.async_copy(dst_hbm.at[t, ch], dst_v.at[slot], isems.at[slot])

    def _wait_idx(ch, slot):
        pltpu.make_async_copy(src_hbm.at[t, ch], src_v.at[slot],
                              isems.at[slot]).wait()
        pltpu.make_async_copy(dst_hbm.at[t, ch], dst_v.at[slot],
                              isems.at[slot]).wait()

    def _start_gather(ch, slot, b):
        pltpu.async_copy(h_hbm.at[src_v.at[slot, 0]], rows.at[b],
                         gsems.at[b])

    for i in range(IB):
        _load_idx(i, i)
    for b in range(NBUF):
        _wait_idx(b, b)
        _start_gather(b, b, b)

    @pl.loop(0, NCH, step=NBUF)
    def _(base):
        for b in range(NBUF):
            ch = base + b

            @pl.when(ch < NCH)
            def _():
                slot = ch % IB
                pltpu.make_async_copy(h_hbm.at[src_v.at[slot, 0]],
                                      rows.at[b], gsems.at[b]).wait()
                pltpu.sync_copy(rows.at[b], acc.at[dst_v.at[slot, 0]],
                                add=True)

                @pl.when(ch + IB < NCH)
                def _():
                    _load_idx(ch + IB, slot)

                @pl.when(ch + NBUF < NCH)
                def _():
                    nslot = (ch + NBUF) % IB
                    _wait_idx(ch + NBUF, nslot)
                    _start_gather(ch + NBUF, nslot, b)

    plsc.subcore_barrier()

    @pl.when(sid < NS - 1)
    def _():
        pltpu.sync_copy(acc.at[pl.ds(row0, RPT_A)],
                        out_hbm.at[cid, pl.ds(row0, RPT_A)])

    @pl.when(sid == NS - 1)
    def _():
        pltpu.sync_copy(acc.at[pl.ds(row0, RPT_B)],
                        out_hbm.at[cid, pl.ds(row0, RPT_B)])


@jax.jit
def _aggregate(h, zeros, src3, dst3):
    mesh = plsc.VectorSubcoreMesh(core_axis_name="core",
                                  subcore_axis_name="subcore")
    return pl.kernel(
        _agg_body,
        out_type=jax.ShapeDtypeStruct((NC, N, D), jnp.float32),
        mesh=mesh,
        scratch_types=[
            pltpu.VMEM_SHARED((N, D), jnp.float32),
            pltpu.VMEM((IB, 1, C), jnp.int32),
            pltpu.VMEM((IB, 1, C), jnp.int32),
            pltpu.VMEM((NBUF, C, D), jnp.float32),
            pltpu.SemaphoreType.DMA((IB,)),
            pltpu.SemaphoreType.DMA((NBUF,)),
        ],
    )(h, zeros, src3, dst3)


def _dense_body(final_relu, decode, refs):
    if decode:
        (p_ref, wa_ref, ba_ref, g_ref, b_ref, wb_ref, bb_ref,
         dw_ref, db_ref, h_ref, o_ref) = refs
    else:
        (p_ref, wa_ref, ba_ref, g_ref, b_ref, wb_ref, bb_ref, h_ref) = refs
    z = p_ref[0] + p_ref[1]
    h1 = jnp.dot(z, wa_ref[...], preferred_element_type=jnp.float32, precision=lax.Precision.HIGHEST)
    h1 = h1 + ba_ref[...]
    m = jnp.mean(h1, axis=0, keepdims=True)
    v = jnp.mean((h1 - m) ** 2, axis=0, keepdims=True)
    hn = (h1 - m) * lax.rsqrt(v + 1e-5) * g_ref[...] + b_ref[...]
    hn = jnp.maximum(hn, 0.0)
    h2 = jnp.dot(hn, wb_ref[...], preferred_element_type=jnp.float32, precision=lax.Precision.HIGHEST)
    h2 = h2 + bb_ref[...]
    if final_relu:
        h2 = jnp.maximum(h2, 0.0)
    h_ref[...] = h2
    if decode:
        o_ref[...] = (jnp.dot(h2, dw_ref[...],
                              preferred_element_type=jnp.float32, precision=lax.Precision.HIGHEST)
                      + db_ref[...])


def _dense(parts, Wa, ba, g, b, Wb, bb, final_relu):
    def body(*refs):
        _dense_body(final_relu, False, refs)
    return pl.pallas_call(
        body,
        out_shape=jax.ShapeDtypeStruct((N, Wb.shape[1]), jnp.float32),
    )(parts, Wa, ba.reshape(1, -1), g.reshape(1, -1), b.reshape(1, -1),
      Wb, bb.reshape(1, -1))


def _dense_decode(parts, Wa, ba, g, b, Wb, bb, dec_W, dec_b):
    def body(*refs):
        _dense_body(False, True, refs)
    return pl.pallas_call(
        body,
        out_shape=[jax.ShapeDtypeStruct((N, Wb.shape[1]), jnp.float32),
                   jax.ShapeDtypeStruct((N, 1), jnp.float32)],
    )(parts, Wa, ba.reshape(1, -1), g.reshape(1, -1), b.reshape(1, -1),
      Wb, bb.reshape(1, -1), dec_W, dec_b.reshape(1, -1))


def kernel(x, edge_index,
           l0_Wa, l0_ba, l0_g, l0_b, l0_Wb, l0_bb,
           l1_Wa, l1_ba, l1_g, l1_b, l1_Wb, l1_bb,
           l2_Wa, l2_ba, l2_g, l2_b, l2_Wb, l2_bb,
           dec_W, dec_b):
    src3 = edge_index[0].astype(jnp.int32).reshape(NT, NCH, 1, C)
    dst3 = edge_index[1].astype(jnp.int32).reshape(NT, NCH, 1, C)
    zeros = jnp.zeros((N, D), jnp.float32)

    h = x
    for Wa, ba, g, b, Wb, bb in [
        (l0_Wa, l0_ba, l0_g, l0_b, l0_Wb, l0_bb),
        (l1_Wa, l1_ba, l1_g, l1_b, l1_Wb, l1_bb),
    ]:
        parts = _aggregate(h, zeros, src3, dst3)
        h = _dense(parts, Wa, ba, g, b, Wb, bb, True)

    parts = _aggregate(h, zeros, src3, dst3)
    h, out = _dense_decode(parts, l2_Wa, l2_ba, l2_g, l2_b, l2_Wb, l2_bb,
                           dec_W, dec_b)
    return (out, h)


# speedup vs baseline: 1.1426x; 1.1426x over previous
import functools

import jax
import jax.numpy as jnp
from jax import lax
from jax.experimental import pallas as pl
from jax.experimental.pallas import tpu as pltpu
from jax.experimental.pallas import tpu_sc as plsc

N = 10000
D = 128
E = 320000

NC = 2
NS = 16
NT = NC * NS
EPT = E // NT
C = 100
NCH = EPT // C
RPT_A = 632
RPT_B = N - (NS - 1) * RPT_A


NBUF = 3
IB = 4


def _agg_body(h_hbm, zeros_hbm, src_hbm, dst_hbm, out_hbm,
              acc, src_v, dst_v, rows, isems, gsems):
    cid = lax.axis_index("core")
    sid = lax.axis_index("subcore")
    t = cid * NS + sid

    row0 = sid * RPT_A
    init_src = [h_hbm, zeros_hbm]
    for c in range(NC):
        @pl.when((cid == c) & (sid < NS - 1))
        def _(src=init_src[c]):
            pltpu.sync_copy(src.at[pl.ds(row0, RPT_A)],
                            acc.at[pl.ds(row0, RPT_A)])

        @pl.when((cid == c) & (sid == NS - 1))
        def _(src=init_src[c]):
            pltpu.sync_copy(src.at[pl.ds(row0, RPT_B)],
                            acc.at[pl.ds(row0, RPT_B)])

    plsc.subcore_barrier()

    def _load_idx(ch, slot):
        pltpu.async_copy(src_hbm.at[t, ch], src_v.at[slot], isems.at[slot])
        pltpu.async_copy(dst_hbm.at[t, ch], dst_v.at[slot], isems.at[slot])

    def _wait_idx(ch, slot):
        pltpu.make_async_copy(src_hbm.at[t, ch], src_v.at[slot],
                              isems.at[slot]).wait()
        pltpu.make_async_copy(dst_hbm.at[t, ch], dst_v.at[slot],
                              isems.at[slot]).wait()

    def _start_gather(ch, slot, b):
        pltpu.async_copy(h_hbm.at[src_v.at[slot]], rows.at[b],
                         gsems.at[b])

    for i in range(IB):
        _load_idx(i, i)
    for b in range(NBUF):
        _wait_idx(b, b)
        _start_gather(b, b, b)

    @pl.loop(0, NCH, step=NBUF)
    def _(base):
        for b in range(NBUF):
            ch = base + b

            @pl.when(ch < NCH)
            def _():
                slot = ch % IB
                pltpu.make_async_copy(h_hbm.at[src_v.at[slot]],
                                      rows.at[b], gsems.at[b]).wait()
                pltpu.sync_copy(rows.at[b], acc.at[dst_v.at[slot]],
                                add=True)

                @pl.when(ch + IB < NCH)
                def _():
                    _load_idx(ch + IB, slot)

                @pl.when(ch + NBUF < NCH)
                def _():
                    nslot = (ch + NBUF) % IB
                    _wait_idx(ch + NBUF, nslot)
                    _start_gather(ch + NBUF, nslot, b)

    plsc.subcore_barrier()

    @pl.when(sid < NS - 1)
    def _():
        pltpu.sync_copy(acc.at[pl.ds(row0, RPT_A)],
                        out_hbm.at[cid, pl.ds(row0, RPT_A)])

    @pl.when(sid == NS - 1)
    def _():
        pltpu.sync_copy(acc.at[pl.ds(row0, RPT_B)],
                        out_hbm.at[cid, pl.ds(row0, RPT_B)])


@jax.jit
def _aggregate(h, zeros, src3, dst3):
    mesh = plsc.VectorSubcoreMesh(core_axis_name="core",
                                  subcore_axis_name="subcore")
    return pl.kernel(
        _agg_body,
        out_type=jax.ShapeDtypeStruct((NC, N, D), jnp.float32),
        mesh=mesh,
        scratch_types=[
            pltpu.VMEM_SHARED((N, D), jnp.float32),
            pltpu.VMEM((IB, C), jnp.int32),
            pltpu.VMEM((IB, C), jnp.int32),
            pltpu.VMEM((NBUF, C, D), jnp.float32),
            pltpu.SemaphoreType.DMA((IB,)),
            pltpu.SemaphoreType.DMA((NBUF,)),
        ],
    )(h, zeros, src3, dst3)


def _dense_body(final_relu, decode, refs):
    if decode:
        (p_ref, wa_ref, ba_ref, g_ref, b_ref, wb_ref, bb_ref,
         dw_ref, db_ref, h_ref, o_ref) = refs
    else:
        (p_ref, wa_ref, ba_ref, g_ref, b_ref, wb_ref, bb_ref, h_ref) = refs
    z = p_ref[0] + p_ref[1]
    h1 = jnp.dot(z, wa_ref[...], preferred_element_type=jnp.float32)
    h1 = h1 + ba_ref[...]
    m = jnp.mean(h1, axis=0, keepdims=True)
    v = jnp.mean((h1 - m) ** 2, axis=0, keepdims=True)
    hn = (h1 - m) * lax.rsqrt(v + 1e-5) * g_ref[...] + b_ref[...]
    hn = jnp.maximum(hn, 0.0)
    h2 = jnp.dot(hn, wb_ref[...], preferred_element_type=jnp.float32)
    h2 = h2 + bb_ref[...]
    if final_relu:
        h2 = jnp.maximum(h2, 0.0)
    h_ref[...] = h2
    if decode:
        o_ref[...] = (jnp.dot(h2, dw_ref[...],
                              preferred_element_type=jnp.float32)
                      + db_ref[...])


def _dense(parts, Wa, ba, g, b, Wb, bb, final_relu):
    def body(*refs):
        _dense_body(final_relu, False, refs)
    return pl.pallas_call(
        body,
        out_shape=jax.ShapeDtypeStruct((N, Wb.shape[1]), jnp.float32),
    )(parts, Wa, ba.reshape(1, -1), g.reshape(1, -1), b.reshape(1, -1),
      Wb, bb.reshape(1, -1))


def _dense_decode(parts, Wa, ba, g, b, Wb, bb, dec_W, dec_b):
    def body(*refs):
        _dense_body(False, True, refs)
    return pl.pallas_call(
        body,
        out_shape=[jax.ShapeDtypeStruct((N, Wb.shape[1]), jnp.float32),
                   jax.ShapeDtypeStruct((N, 1), jnp.float32)],
    )(parts, Wa, ba.reshape(1, -1), g.reshape(1, -1), b.reshape(1, -1),
      Wb, bb.reshape(1, -1), dec_W, dec_b.reshape(1, -1))


def kernel(x, edge_index,
           l0_Wa, l0_ba, l0_g, l0_b, l0_Wb, l0_bb,
           l1_Wa, l1_ba, l1_g, l1_b, l1_Wb, l1_bb,
           l2_Wa, l2_ba, l2_g, l2_b, l2_Wb, l2_bb,
           dec_W, dec_b):
    src3 = edge_index[0].astype(jnp.int32).reshape(NT, NCH, C)
    dst3 = edge_index[1].astype(jnp.int32).reshape(NT, NCH, C)
    zeros = jnp.zeros((N, D), jnp.float32)

    h = x
    for Wa, ba, g, b, Wb, bb in [
        (l0_Wa, l0_ba, l0_g, l0_b, l0_Wb, l0_bb),
        (l1_Wa, l1_ba, l1_g, l1_b, l1_Wb, l1_bb),
    ]:
        parts = _aggregate(h, zeros, src3, dst3)
        h = _dense(parts, Wa, ba, g, b, Wb, bb, True)

    parts = _aggregate(h, zeros, src3, dst3)
    h, out = _dense_decode(parts, l2_Wa, l2_ba, l2_g, l2_b, l2_Wb, l2_bb,
                           dec_W, dec_b)
    return (out, h)
